# baseline (device time: 40802 ns/iter reference)
import functools

import jax
import jax.numpy as jnp
from jax import lax
from jax.experimental import pallas as pl
from jax.experimental.pallas import tpu as pltpu

N_DEV = 4
SEQ = 1024
HALO = 128
EXT = SEQ + 2 * HALO
HQ = 8
DH = 128
D = HQ * DH
WINDOW = 128
SCALE = 0.08838834764831843


def kernel(x, Wq, K_ext, V_ext, Wo):
    x2 = x.reshape(SEQ, D)
    K3 = K_ext.reshape(SEQ, HQ, DH)
    V3 = V_ext.reshape(SEQ, HQ, DH)

    def body(x_ref, wq_ref, k_ref, v_ref, wo_ref, out_ref,
             ext_k, ext_v, ctx_ref, sbuf, rbuf, send_sems, recv_sems):
        my = lax.axis_index("i")
        left = (my - 1) % N_DEV
        right = (my + 1) % N_DEV

        barrier = pltpu.get_barrier_semaphore()
        for nbr in (left, right):
            pl.semaphore_signal(barrier, inc=1, device_id=(nbr,),
                                device_id_type=pl.DeviceIdType.MESH,)
        pl.semaphore_wait(barrier, 2)

        def rdma(si, to):
            return pltpu.make_async_remote_copy(
                src_ref=sbuf.at[si],
                dst_ref=rbuf.at[si],
                send_sem=send_sems.at[si],
                recv_sem=recv_sems.at[si],
                device_id=(to,),
                device_id_type=pl.DeviceIdType.MESH,
            )

        @pl.when(my < N_DEV - 1)
        def _():
            for h in range(HQ):
                sbuf[0, h, :, :] = k_ref[SEQ - HALO:SEQ, h, :].astype(jnp.bfloat16)
                sbuf[1, h, :, :] = v_ref[SEQ - HALO:SEQ, h, :].astype(jnp.bfloat16)
            rdma(0, right).start()
            rdma(1, right).start()

        @pl.when(my > 0)
        def _():
            for h in range(HQ):
                sbuf[2, h, :, :] = k_ref[0:HALO, h, :].astype(jnp.bfloat16)
                sbuf[3, h, :, :] = v_ref[0:HALO, h, :].astype(jnp.bfloat16)
            rdma(2, left).start()
            rdma(3, left).start()

        for h in range(HQ):
            ext_k[h, pl.ds(HALO, SEQ), :] = k_ref[:, h, :].astype(jnp.bfloat16)
            ext_v[h, pl.ds(HALO, SEQ), :] = v_ref[:, h, :].astype(jnp.bfloat16)

        zeros = jnp.zeros((HQ, HALO, DH), jnp.bfloat16)

        @pl.when(my == 0)
        def _():
            ext_k[:, pl.ds(0, HALO), :] = zeros
            ext_v[:, pl.ds(0, HALO), :] = zeros

        @pl.when(my == N_DEV - 1)
        def _():
            ext_k[:, pl.ds(EXT - HALO, HALO), :] = zeros
            ext_v[:, pl.ds(EXT - HALO, HALO), :] = zeros

        q = (jnp.dot(x_ref[:, :].astype(jnp.bfloat16),
                     wq_ref[:, :].astype(jnp.bfloat16),
                     preferred_element_type=jnp.float32)
             * SCALE).astype(jnp.bfloat16)

        QB = 256
        KW = QB + 2 * HALO
        N_QB = SEQ // QB

        def attn_block(qb):
            r = lax.broadcasted_iota(jnp.int32, (QB, KW), 0)
            c = lax.broadcasted_iota(jnp.int32, (QB, KW), 1)
            kg = my * SEQ - HALO + qb * QB + c
            valid = (jnp.abs(r + HALO - c) <= WINDOW) \
                & (kg >= 0) & (kg < N_DEV * SEQ)
            bias = jnp.where(valid, 0.0, -1e9).astype(jnp.float32)
            for h in range(HQ):
                qh = q[qb * QB:(qb + 1) * QB, h * DH:(h + 1) * DH]
                kh = ext_k[h, pl.ds(qb * QB, KW), :]
                vh = ext_v[h, pl.ds(qb * QB, KW), :]
                s = lax.dot_general(
                    qh, kh, (((1,), (1,)), ((), ())),
                    preferred_element_type=jnp.float32,
                ) + bias
                w = jnp.exp(s)
                inv = 1.0 / jnp.sum(w, axis=1, keepdims=True)
                ctx_ref[pl.ds(qb * QB, QB), pl.ds(h * DH, DH)] = (jnp.dot(
                    w.astype(jnp.bfloat16), vh,
                    preferred_element_type=jnp.float32) * inv
                ).astype(jnp.bfloat16)

        attn_block(1)
        attn_block(2)

        @pl.when(my > 0)
        def _():
            rdma(0, left).wait_recv()
            rdma(1, left).wait_recv()
            for h in range(HQ):
                ext_k[h, pl.ds(0, HALO), :] = rbuf[0, h, :, :]
                ext_v[h, pl.ds(0, HALO), :] = rbuf[1, h, :, :]

        attn_block(0)

        @pl.when(my < N_DEV - 1)
        def _():
            rdma(2, right).wait_recv()
            rdma(3, right).wait_recv()
            for h in range(HQ):
                ext_k[h, pl.ds(EXT - HALO, HALO), :] = rbuf[2, h, :, :]
                ext_v[h, pl.ds(EXT - HALO, HALO), :] = rbuf[3, h, :, :]

        attn_block(N_QB - 1)

        out_ref[:, :] = jnp.dot(ctx_ref[:, :],
                                wo_ref[:, :].astype(jnp.bfloat16),
                                preferred_element_type=jnp.float32)

        @pl.when(my < N_DEV - 1)
        def _():
            rdma(0, right).wait_send()
            rdma(1, right).wait_send()

        @pl.when(my > 0)
        def _():
            rdma(2, left).wait_send()
            rdma(3, left).wait_send()

        @functools.partial(pl.run_scoped, sem2=pltpu.SemaphoreType.REGULAR)
        def _(sem2):
            for nbr in (left, right):
                pl.semaphore_signal(sem2, inc=1, device_id=(nbr,),
                                    device_id_type=pl.DeviceIdType.MESH)
            pl.semaphore_wait(sem2, 2)

    out = pl.pallas_call(
        body,
        out_shape=jax.ShapeDtypeStruct((SEQ, D), jnp.float32),
        in_specs=[pl.BlockSpec(memory_space=pltpu.VMEM)] * 5,
        out_specs=pl.BlockSpec(memory_space=pltpu.VMEM),
        scratch_shapes=[
            pltpu.VMEM((HQ, EXT, DH), jnp.bfloat16),
            pltpu.VMEM((HQ, EXT, DH), jnp.bfloat16),
            pltpu.VMEM((SEQ, D), jnp.bfloat16),
            pltpu.VMEM((4, HQ, HALO, DH), jnp.bfloat16),
            pltpu.VMEM((4, HQ, HALO, DH), jnp.bfloat16),
            pltpu.SemaphoreType.DMA((4,)),
            pltpu.SemaphoreType.DMA((4,)),
        ],
        compiler_params=pltpu.CompilerParams(collective_id=0),
    )(x2, Wq, K3, V3, Wo)
    return out.reshape(1, SEQ, D)


# device time: 39212 ns/iter; 1.0405x vs baseline; 1.0405x over previous
import functools

import jax
import jax.numpy as jnp
from jax import lax
from jax.experimental import pallas as pl
from jax.experimental.pallas import tpu as pltpu

N_DEV = 4
SEQ = 1024
HALO = 128
EXT = SEQ + 2 * HALO
HQ = 8
DH = 128
D = HQ * DH
WINDOW = 128
SCALE = 0.08838834764831843


def kernel(x, Wq, K_ext, V_ext, Wo):
    x2 = x.reshape(SEQ, D)
    K3 = K_ext.reshape(SEQ, HQ, DH)
    V3 = V_ext.reshape(SEQ, HQ, DH)

    def body(x_ref, wq_ref, k_ref, v_ref, wo_ref, out_ref,
             flat_k, flat_v, ext_k, ext_v, ctx_ref, sbuf,
             dma_sems, send_sems, recv_sems):
        my = lax.axis_index("i")
        left = (my - 1) % N_DEV
        right = (my + 1) % N_DEV

        barrier = pltpu.get_barrier_semaphore()
        for nbr in (left, right):
            pl.semaphore_signal(barrier, inc=1, device_id=(nbr,),
                                device_id_type=pl.DeviceIdType.MESH)
        pl.semaphore_wait(barrier, 2)

        def relayout(src, dst, ti):
            return [
                pltpu.make_async_copy(
                    src.at[:, h, :],
                    dst.at[:, pl.ds(h * DH, DH)],
                    dma_sems.at[ti, h],
                )
                for h in range(HQ)
            ]

        k_dmas = relayout(k_ref, flat_k, 0)
        v_dmas = relayout(v_ref, flat_v, 1)
        for c in k_dmas + v_dmas:
            c.start()

        q = (jnp.dot(x_ref[:, :].astype(jnp.bfloat16),
                     wq_ref[:, :].astype(jnp.bfloat16),
                     preferred_element_type=jnp.float32)
             * SCALE).astype(jnp.bfloat16)

        for c in k_dmas + v_dmas:
            c.wait()

        def rdma(si, ext_ref, dst_off, to):
            return pltpu.make_async_remote_copy(
                src_ref=sbuf.at[si],
                dst_ref=ext_ref.at[pl.ds(dst_off, HALO)],
                send_sem=send_sems.at[si],
                recv_sem=recv_sems.at[si],
                device_id=(to,),
                device_id_type=pl.DeviceIdType.MESH,
            )

        @pl.when(my < N_DEV - 1)
        def _():
            sbuf[0, :, :] = flat_k[SEQ - HALO:SEQ, :].astype(jnp.bfloat16)
            sbuf[1, :, :] = flat_v[SEQ - HALO:SEQ, :].astype(jnp.bfloat16)
            rdma(0, ext_k, 0, right).start()
            rdma(1, ext_v, 0, right).start()

        @pl.when(my > 0)
        def _():
            sbuf[2, :, :] = flat_k[0:HALO, :].astype(jnp.bfloat16)
            sbuf[3, :, :] = flat_v[0:HALO, :].astype(jnp.bfloat16)
            rdma(2, ext_k, EXT - HALO, left).start()
            rdma(3, ext_v, EXT - HALO, left).start()

        ext_k[pl.ds(HALO, SEQ), :] = flat_k[:, :].astype(jnp.bfloat16)
        ext_v[pl.ds(HALO, SEQ), :] = flat_v[:, :].astype(jnp.bfloat16)

        zeros = jnp.zeros((HALO, D), jnp.bfloat16)

        @pl.when(my == 0)
        def _():
            ext_k[pl.ds(0, HALO), :] = zeros
            ext_v[pl.ds(0, HALO), :] = zeros

        @pl.when(my == N_DEV - 1)
        def _():
            ext_k[pl.ds(EXT - HALO, HALO), :] = zeros
            ext_v[pl.ds(EXT - HALO, HALO), :] = zeros

        QB = 256
        KW = QB + 2 * HALO
        N_QB = SEQ // QB

        def attn_block(qb):
            r = lax.broadcasted_iota(jnp.int32, (QB, KW), 0)
            c = lax.broadcasted_iota(jnp.int32, (QB, KW), 1)
            kg = my * SEQ - HALO + qb * QB + c
            valid = (jnp.abs(r + HALO - c) <= WINDOW) \
                & (kg >= 0) & (kg < N_DEV * SEQ)
            bias = jnp.where(valid, 0.0, -1e9).astype(jnp.float32)
            for h in range(HQ):
                qh = q[qb * QB:(qb + 1) * QB, h * DH:(h + 1) * DH]
                kh = ext_k[pl.ds(qb * QB, KW), pl.ds(h * DH, DH)]
                vh = ext_v[pl.ds(qb * QB, KW), pl.ds(h * DH, DH)]
                s = lax.dot_general(
                    qh, kh, (((1,), (1,)), ((), ())),
                    preferred_element_type=jnp.float32,
                ) + bias
                w = jnp.exp(s)
                inv = 1.0 / jnp.sum(w, axis=1, keepdims=True)
                ctx_ref[pl.ds(qb * QB, QB), pl.ds(h * DH, DH)] = (jnp.dot(
                    w.astype(jnp.bfloat16), vh,
                    preferred_element_type=jnp.float32) * inv
                ).astype(jnp.bfloat16)

        attn_block(1)
        attn_block(2)

        @pl.when(my > 0)
        def _():
            rdma(0, ext_k, 0, left).wait_recv()
            rdma(1, ext_v, 0, left).wait_recv()

        attn_block(0)

        @pl.when(my < N_DEV - 1)
        def _():
            rdma(2, ext_k, EXT - HALO, right).wait_recv()
            rdma(3, ext_v, EXT - HALO, right).wait_recv()

        attn_block(N_QB - 1)

        out_ref[:, :] = jnp.dot(ctx_ref[:, :],
                                wo_ref[:, :].astype(jnp.bfloat16),
                                preferred_element_type=jnp.float32)

        @pl.when(my < N_DEV - 1)
        def _():
            rdma(0, ext_k, 0, right).wait_send()
            rdma(1, ext_v, 0, right).wait_send()

        @pl.when(my > 0)
        def _():
            rdma(2, ext_k, EXT - HALO, left).wait_send()
            rdma(3, ext_v, EXT - HALO, left).wait_send()

        @functools.partial(pl.run_scoped, sem2=pltpu.SemaphoreType.REGULAR)
        def _(sem2):
            for nbr in (left, right):
                pl.semaphore_signal(sem2, inc=1, device_id=(nbr,),
                                    device_id_type=pl.DeviceIdType.MESH)
            pl.semaphore_wait(sem2, 2)

    out = pl.pallas_call(
        body,
        out_shape=jax.ShapeDtypeStruct((SEQ, D), jnp.float32),
        in_specs=[pl.BlockSpec(memory_space=pltpu.VMEM)] * 5,
        out_specs=pl.BlockSpec(memory_space=pltpu.VMEM),
        scratch_shapes=[
            pltpu.VMEM((SEQ, D), jnp.float32),
            pltpu.VMEM((SEQ, D), jnp.float32),
            pltpu.VMEM((EXT, D), jnp.bfloat16),
            pltpu.VMEM((EXT, D), jnp.bfloat16),
            pltpu.VMEM((SEQ, D), jnp.bfloat16),
            pltpu.VMEM((4, HALO, D), jnp.bfloat16),
            pltpu.SemaphoreType.DMA((2, HQ)),
            pltpu.SemaphoreType.DMA((4,)),
            pltpu.SemaphoreType.DMA((4,)),
        ],
        compiler_params=pltpu.CompilerParams(collective_id=0),
    )(x2, Wq, K3, V3, Wo)
    return out.reshape(1, SEQ, D)


# device time: 35371 ns/iter; 1.1535x vs baseline; 1.1086x over previous
import functools

import jax
import jax.numpy as jnp
from jax import lax
from jax.experimental import pallas as pl
from jax.experimental.pallas import tpu as pltpu

N_DEV = 4
SEQ = 1024
HALO = 128
EXT = SEQ + 2 * HALO
HQ = 8
DH = 128
D = HQ * DH
WINDOW = 128
SCALE = 0.08838834764831843


def kernel(x, Wq, K_ext, V_ext, Wo):
    x2 = x.reshape(SEQ, D)
    K3 = K_ext.reshape(SEQ, HQ, DH)
    V3 = V_ext.reshape(SEQ, HQ, DH)

    def body(x_ref, wq_ref, k_ref, v_ref, wo_ref, out_ref,
             flat_k, flat_v, ext_k, ext_v, ctx_ref, sbuf,
             dma_sems, send_sems, recv_sems):
        my = lax.axis_index("i")
        left = (my - 1) % N_DEV
        right = (my + 1) % N_DEV

        barrier = pltpu.get_barrier_semaphore()
        for nbr in (left, right):
            pl.semaphore_signal(barrier, inc=1, device_id=(nbr,),
                                device_id_type=pl.DeviceIdType.MESH)
        pl.semaphore_wait(barrier, 2)

        flat_k[:, :] = k_ref[:, :, :].reshape(SEQ, D)
        flat_v[:, :] = v_ref[:, :, :].reshape(SEQ, D)

        q = (jnp.dot(x_ref[:, :].astype(jnp.bfloat16),
                     wq_ref[:, :].astype(jnp.bfloat16),
                     preferred_element_type=jnp.float32)
             * SCALE).astype(jnp.bfloat16)

        def rdma(si, ext_ref, dst_off, to):
            return pltpu.make_async_remote_copy(
                src_ref=sbuf.at[si],
                dst_ref=ext_ref.at[pl.ds(dst_off, HALO)],
                send_sem=send_sems.at[si],
                recv_sem=recv_sems.at[si],
                device_id=(to,),
                device_id_type=pl.DeviceIdType.MESH,
            )

        @pl.when(my < N_DEV - 1)
        def _():
            sbuf[0, :, :] = flat_k[SEQ - HALO:SEQ, :].astype(jnp.bfloat16)
            sbuf[1, :, :] = flat_v[SEQ - HALO:SEQ, :].astype(jnp.bfloat16)
            rdma(0, ext_k, 0, right).start()
            rdma(1, ext_v, 0, right).start()

        @pl.when(my > 0)
        def _():
            sbuf[2, :, :] = flat_k[0:HALO, :].astype(jnp.bfloat16)
            sbuf[3, :, :] = flat_v[0:HALO, :].astype(jnp.bfloat16)
            rdma(2, ext_k, EXT - HALO, left).start()
            rdma(3, ext_v, EXT - HALO, left).start()

        ext_k[pl.ds(HALO, SEQ), :] = flat_k[:, :].astype(jnp.bfloat16)
        ext_v[pl.ds(HALO, SEQ), :] = flat_v[:, :].astype(jnp.bfloat16)

        zeros = jnp.zeros((HALO, D), jnp.bfloat16)

        @pl.when(my == 0)
        def _():
            ext_k[pl.ds(0, HALO), :] = zeros
            ext_v[pl.ds(0, HALO), :] = zeros

        @pl.when(my == N_DEV - 1)
        def _():
            ext_k[pl.ds(EXT - HALO, HALO), :] = zeros
            ext_v[pl.ds(EXT - HALO, HALO), :] = zeros

        QB = 256
        KW = QB + 2 * HALO
        N_QB = SEQ // QB

        def attn_block(qb):
            r = lax.broadcasted_iota(jnp.int32, (QB, KW), 0)
            c = lax.broadcasted_iota(jnp.int32, (QB, KW), 1)
            kg = my * SEQ - HALO + qb * QB + c
            valid = (jnp.abs(r + HALO - c) <= WINDOW) \
                & (kg >= 0) & (kg < N_DEV * SEQ)
            bias = jnp.where(valid, 0.0, -1e9).astype(jnp.float32)
            for h in range(HQ):
                qh = q[qb * QB:(qb + 1) * QB, h * DH:(h + 1) * DH]
                kh = ext_k[pl.ds(qb * QB, KW), pl.ds(h * DH, DH)]
                vh = ext_v[pl.ds(qb * QB, KW), pl.ds(h * DH, DH)]
                s = lax.dot_general(
                    qh, kh, (((1,), (1,)), ((), ())),
                    preferred_element_type=jnp.float32,
                ) + bias
                w = jnp.exp(s)
                inv = 1.0 / jnp.sum(w, axis=1, keepdims=True)
                ctx_ref[pl.ds(qb * QB, QB), pl.ds(h * DH, DH)] = (jnp.dot(
                    w.astype(jnp.bfloat16), vh,
                    preferred_element_type=jnp.float32) * inv
                ).astype(jnp.bfloat16)

        attn_block(1)
        attn_block(2)

        @pl.when(my > 0)
        def _():
            rdma(0, ext_k, 0, left).wait_recv()
            rdma(1, ext_v, 0, left).wait_recv()

        attn_block(0)

        @pl.when(my < N_DEV - 1)
        def _():
            rdma(2, ext_k, EXT - HALO, right).wait_recv()
            rdma(3, ext_v, EXT - HALO, right).wait_recv()

        attn_block(N_QB - 1)

        out_ref[:, :] = jnp.dot(ctx_ref[:, :],
                                wo_ref[:, :].astype(jnp.bfloat16),
                                preferred_element_type=jnp.float32)

        @pl.when(my < N_DEV - 1)
        def _():
            rdma(0, ext_k, 0, right).wait_send()
            rdma(1, ext_v, 0, right).wait_send()

        @pl.when(my > 0)
        def _():
            rdma(2, ext_k, EXT - HALO, left).wait_send()
            rdma(3, ext_v, EXT - HALO, left).wait_send()

        @functools.partial(pl.run_scoped, sem2=pltpu.SemaphoreType.REGULAR)
        def _(sem2):
            for nbr in (left, right):
                pl.semaphore_signal(sem2, inc=1, device_id=(nbr,),
                                    device_id_type=pl.DeviceIdType.MESH)
            pl.semaphore_wait(sem2, 2)

    out = pl.pallas_call(
        body,
        out_shape=jax.ShapeDtypeStruct((SEQ, D), jnp.float32),
        in_specs=[pl.BlockSpec(memory_space=pltpu.VMEM)] * 5,
        out_specs=pl.BlockSpec(memory_space=pltpu.VMEM),
        scratch_shapes=[
            pltpu.VMEM((SEQ, D), jnp.float32),
            pltpu.VMEM((SEQ, D), jnp.float32),
            pltpu.VMEM((EXT, D), jnp.bfloat16),
            pltpu.VMEM((EXT, D), jnp.bfloat16),
            pltpu.VMEM((SEQ, D), jnp.bfloat16),
            pltpu.VMEM((4, HALO, D), jnp.bfloat16),
            pltpu.SemaphoreType.DMA((2, HQ)),
            pltpu.SemaphoreType.DMA((4,)),
            pltpu.SemaphoreType.DMA((4,)),
        ],
        compiler_params=pltpu.CompilerParams(collective_id=0),
    )(x2, Wq, K3, V3, Wo)
    return out.reshape(1, SEQ, D)


# device time: 27852 ns/iter; 1.4650x vs baseline; 1.2700x over previous
import functools

import jax
import jax.numpy as jnp
from jax import lax
from jax.experimental import pallas as pl
from jax.experimental.pallas import tpu as pltpu

N_DEV = 4
SEQ = 1024
HALO = 128
EXT = SEQ + 2 * HALO
HQ = 8
DH = 128
D = HQ * DH
WINDOW = 128
SCALE = 0.08838834764831843


def kernel(x, Wq, K_ext, V_ext, Wo):
    x2 = x.reshape(SEQ, D)
    K2 = K_ext.reshape(SEQ, D).astype(jnp.bfloat16)
    V2 = V_ext.reshape(SEQ, D).astype(jnp.bfloat16)

    def body(x_ref, wq_ref, k_ref, v_ref, wo_ref, out_ref,
             ext_k, ext_v, ctx_ref, send_sems, recv_sems):
        my = lax.axis_index("i")
        left = (my - 1) % N_DEV
        right = (my + 1) % N_DEV

        barrier = pltpu.get_barrier_semaphore()
        for nbr in (left, right):
            pl.semaphore_signal(barrier, inc=1, device_id=(nbr,),
                                device_id_type=pl.DeviceIdType.MESH)
        pl.semaphore_wait(barrier, 2)

        def rdma(si, src, src_off, ext_ref, dst_off, to):
            return pltpu.make_async_remote_copy(
                src_ref=src.at[pl.ds(src_off, HALO)],
                dst_ref=ext_ref.at[pl.ds(dst_off, HALO)],
                send_sem=send_sems.at[si],
                recv_sem=recv_sems.at[si],
                device_id=(to,),
                device_id_type=pl.DeviceIdType.MESH,
            )

        def rdma_rightward(si, src, ext_ref, to):
            return rdma(si, src, SEQ - HALO, ext_ref, 0, to)

        def rdma_leftward(si, src, ext_ref, to):
            return rdma(si, src, 0, ext_ref, EXT - HALO, to)

        @pl.when(my < N_DEV - 1)
        def _():
            rdma_rightward(0, k_ref, ext_k, right).start()
            rdma_rightward(1, v_ref, ext_v, right).start()

        @pl.when(my > 0)
        def _():
            rdma_leftward(2, k_ref, ext_k, left).start()
            rdma_leftward(3, v_ref, ext_v, left).start()

        ext_k[pl.ds(HALO, SEQ), :] = k_ref[:, :]
        ext_v[pl.ds(HALO, SEQ), :] = v_ref[:, :]

        zeros = jnp.zeros((HALO, D), jnp.bfloat16)

        @pl.when(my == 0)
        def _():
            ext_k[pl.ds(0, HALO), :] = zeros
            ext_v[pl.ds(0, HALO), :] = zeros

        @pl.when(my == N_DEV - 1)
        def _():
            ext_k[pl.ds(EXT - HALO, HALO), :] = zeros
            ext_v[pl.ds(EXT - HALO, HALO), :] = zeros

        q = (jnp.dot(x_ref[:, :].astype(jnp.bfloat16),
                     wq_ref[:, :].astype(jnp.bfloat16),
                     preferred_element_type=jnp.float32)
             * SCALE).astype(jnp.bfloat16)

        QB = 256
        KW = QB + 2 * HALO
        N_QB = SEQ // QB

        r = lax.broadcasted_iota(jnp.int32, (QB, KW), 0)
        c = lax.broadcasted_iota(jnp.int32, (QB, KW), 1)
        base_bias = jnp.where(jnp.abs(r + HALO - c) <= WINDOW, 0.0, -1e9)
        bias_first = jnp.where((my == 0) & (c < HALO), -1e9, base_bias)
        bias_last = jnp.where((my == N_DEV - 1) & (c >= KW - HALO),
                              -1e9, base_bias)

        def attn_block(qb, bias):
            for h in range(HQ):
                qh = q[qb * QB:(qb + 1) * QB, h * DH:(h + 1) * DH]
                kh = ext_k[pl.ds(qb * QB, KW), pl.ds(h * DH, DH)]
                vh = ext_v[pl.ds(qb * QB, KW), pl.ds(h * DH, DH)]
                s = lax.dot_general(
                    qh, kh, (((1,), (1,)), ((), ())),
                    preferred_element_type=jnp.float32,
                ) + bias
                w = jnp.exp(s)
                inv = 1.0 / jnp.sum(w, axis=1, keepdims=True)
                ctx_ref[pl.ds(qb * QB, QB), pl.ds(h * DH, DH)] = (jnp.dot(
                    w.astype(jnp.bfloat16), vh,
                    preferred_element_type=jnp.float32) * inv
                ).astype(jnp.bfloat16)

        attn_block(1, base_bias)
        attn_block(2, base_bias)

        @pl.when(my > 0)
        def _():
            rdma_rightward(0, k_ref, ext_k, left).wait_recv()
            rdma_rightward(1, v_ref, ext_v, left).wait_recv()

        attn_block(0, bias_first)

        @pl.when(my < N_DEV - 1)
        def _():
            rdma_leftward(2, k_ref, ext_k, right).wait_recv()
            rdma_leftward(3, v_ref, ext_v, right).wait_recv()

        attn_block(N_QB - 1, bias_last)

        out_ref[:, :] = jnp.dot(ctx_ref[:, :],
                                wo_ref[:, :].astype(jnp.bfloat16),
                                preferred_element_type=jnp.float32)

        @pl.when(my < N_DEV - 1)
        def _():
            rdma_rightward(0, k_ref, ext_k, right).wait_send()
            rdma_rightward(1, v_ref, ext_v, right).wait_send()

        @pl.when(my > 0)
        def _():
            rdma_leftward(2, k_ref, ext_k, left).wait_send()
            rdma_leftward(3, v_ref, ext_v, left).wait_send()

        @functools.partial(pl.run_scoped, sem2=pltpu.SemaphoreType.REGULAR)
        def _(sem2):
            for nbr in (left, right):
                pl.semaphore_signal(sem2, inc=1, device_id=(nbr,),
                                    device_id_type=pl.DeviceIdType.MESH)
            pl.semaphore_wait(sem2, 2)

    out = pl.pallas_call(
        body,
        out_shape=jax.ShapeDtypeStruct((SEQ, D), jnp.float32),
        in_specs=[pl.BlockSpec(memory_space=pltpu.VMEM)] * 5,
        out_specs=pl.BlockSpec(memory_space=pltpu.VMEM),
        scratch_shapes=[
            pltpu.VMEM((EXT, D), jnp.bfloat16),
            pltpu.VMEM((EXT, D), jnp.bfloat16),
            pltpu.VMEM((SEQ, D), jnp.bfloat16),
            pltpu.SemaphoreType.DMA((4,)),
            pltpu.SemaphoreType.DMA((4,)),
        ],
        compiler_params=pltpu.CompilerParams(collective_id=0),
    )(x2, Wq, K2, V2, Wo)
    return out.reshape(1, SEQ, D)
